# baseline (speedup 1.0000x reference)
import functools
import math

import numpy as np
import jax
import jax.numpy as jnp
from jax import lax
from jax.experimental import pallas as pl
from jax.experimental.pallas import tpu as pltpu
from jax.experimental.pallas import tpu_sc as plsc

_HEADS = 8
_KD = 512
_HALF = 256
_NK = 512
_KNN = 32
_IN = 1024
_VD = 256
_BS = 2048
_TB = 256
_NB = _BS // _TB
_NCAND = 128
_NW = 32
_TPW = _BS // _NW


def _build_cand_tables():
    ci, cj = [], []
    for i in range(_KNN):
        for j in range(min(_KNN, _KNN // (i + 1))):
            ci.append(i)
            cj.append(j)
    n = len(ci)
    assert n <= _NCAND
    p1 = np.zeros((_KNN, _NCAND), np.float32)
    p2 = np.zeros((_KNN, _NCAND), np.float32)
    for c in range(n):
        p1[ci[c], c] = 1.0
        p2[cj[c], c] = 1.0
    bias = np.zeros((_NCAND, 1), np.float32)
    bias[n:, 0] = -1e30
    return p1, p2, bias


_P1, _P2, _BIAS = _build_cand_tables()


def _topk32_t(s, src=None):
    c, t = s.shape
    io = lax.broadcasted_iota(jnp.int32, (c, t), 0)
    lane = lax.broadcasted_iota(jnp.int32, (_KNN, t), 0)
    outv = jnp.zeros((_KNN, t), jnp.float32)
    outi = jnp.zeros((_KNN, t), jnp.float32)
    neg = jnp.float32(float("-inf"))
    for k in range(_KNN):
        m = jnp.max(s, axis=0, keepdims=True)
        am = jnp.min(jnp.where(s == m, io, c), axis=0, keepdims=True)
        hit = io == am
        if src is None:
            pick = am.astype(jnp.float32)
        else:
            pick = jnp.sum(jnp.where(hit, src, 0.0), axis=0, keepdims=True)
        outv = jnp.where(lane == k, m, outv)
        outi = jnp.where(lane == k, pick, outi)
        s = jnp.where(hit, neg, s)
    return outv, outi


def _tc_body(x_ref, wq_ref, bq_ref, keys_ref, p1_ref, p2_ref, bias_ref,
             eye_ref, sc_out_ref, idx_out_ref, sc_scr, idx_scr):
    h = pl.program_id(1)
    hi = lax.Precision.HIGHEST
    q = jnp.dot(x_ref[...], wq_ref[...],
                preferred_element_type=jnp.float32)
    q = q + bq_ref[0]
    k1 = keys_ref[0, 0]
    k2 = keys_ref[0, 1]
    dn = (((1,), (1,)), ((), ()))
    dt = (((1,), (1,)), ((), ()))
    s1t = lax.dot_general(k1, q[:, :_HALF], dt,
                          preferred_element_type=jnp.float32)
    s2t = lax.dot_general(k2, q[:, _HALF:], dt,
                          preferred_element_type=jnp.float32)
    v1t, i1t = _topk32_t(s1t)
    v2t, i2t = _topk32_t(s2t)
    p1 = p1_ref[...]
    p2 = p2_ref[...]
    d0 = (((0,), (0,)), ((), ()))
    cst = lax.dot_general(p1, v1t, d0, precision=hi,
                          preferred_element_type=jnp.float32) \
        + lax.dot_general(p2, v2t, d0, precision=hi,
                          preferred_element_type=jnp.float32) \
        + bias_ref[...]
    cidxt = lax.dot_general(p1, i1t, d0, precision=hi,
                            preferred_element_type=jnp.float32) \
        * jnp.float32(_NK) \
        + lax.dot_general(p2, i2t, d0, precision=hi,
                          preferred_element_type=jnp.float32)
    svt, sit = _topk32_t(cst, src=cidxt)
    e = jnp.exp(svt - svt[0:1])
    smt = e / jnp.sum(e, axis=0, keepdims=True)
    eye = eye_ref[...]
    sm = lax.dot_general(smt, eye, d0, precision=hi,
                         preferred_element_type=jnp.float32)
    si = lax.dot_general(sit, eye, d0, precision=hi,
                         preferred_element_type=jnp.float32)
    sc_scr[h] = sm
    idx_scr[h] = (si + jnp.float32(0.5)).astype(jnp.int32)

    @pl.when(h == _HEADS - 1)
    def _flush():
        for p in range(2):
            sc_out_ref[p] = jnp.concatenate(
                [sc_scr[i] for i in range(4 * p, 4 * p + 4)], axis=1)
        for k in range(4):
            v = jnp.concatenate([idx_scr[2 * k], idx_scr[2 * k + 1]], axis=1)
            p0 = v + (v & jnp.int32(-8))
            idx_out_ref[k] = jnp.concatenate([p0, p0 + jnp.int32(8)], axis=1)


def _tc_topk(x, w_q, bq2, keys, p1, p2, bias):
    f32 = jnp.float32
    return pl.pallas_call(
        _tc_body,
        grid=(_NB, _HEADS),
        in_specs=[
            pl.BlockSpec((_TB, _IN), lambda b, h: (b, 0)),
            pl.BlockSpec((_IN, _KD), lambda b, h: (0, h)),
            pl.BlockSpec((1, 1, _KD), lambda b, h: (h, 0, 0)),
            pl.BlockSpec((1, 2, _NK, _HALF), lambda b, h: (h, 0, 0, 0)),
            pl.BlockSpec((_KNN, _NCAND), lambda b, h: (0, 0)),
            pl.BlockSpec((_KNN, _NCAND), lambda b, h: (0, 0)),
            pl.BlockSpec((_NCAND, 1), lambda b, h: (0, 0)),
            pl.BlockSpec((_KNN, _KNN), lambda b, h: (0, 0)),
        ],
        out_specs=[
            pl.BlockSpec((2, _TB, 128), lambda b, h: (0, b, 0)),
            pl.BlockSpec((4, _TB, 128), lambda b, h: (0, b, 0)),
        ],
        out_shape=[
            jax.ShapeDtypeStruct((2, _BS, 128), f32),
            jax.ShapeDtypeStruct((4, _BS, 128), jnp.int32),
        ],
        scratch_shapes=[
            pltpu.VMEM((_HEADS, _TB, _KNN), f32),
            pltpu.VMEM((_HEADS, _TB, _KNN), jnp.int32),
        ],
    )(x, w_q, bq2, keys, p1, p2, bias, jnp.eye(_KNN, dtype=f32))


def _acc_chunk(buf, acc):
    def body_lo(r, a):
        r4 = r * 4
        for rr in range(4):
            a = tuple(
                (a[c] + buf[r4 + rr, pl.ds(c * 16, 16)]) if c < 8 else a[c]
                for c in range(16)
            )
        return a

    def body_hi(r, a):
        r4 = 64 + r * 4
        for rr in range(4):
            a = tuple(
                (a[c] + buf[r4 + rr, pl.ds((c - 8) * 16, 16)]) if c >= 8
                else a[c]
                for c in range(16)
            )
        return a

    acc = lax.fori_loop(0, 16, body_lo, acc)
    return lax.fori_loop(0, 16, body_hi, acc)


def _sc_body(values_hbm, idx_hbm, sc_hbm, out_hbm, idx_v, sc_v, buf0, buf1,
             res_v, sem0, sem1):
    nc = 2
    wid = lax.axis_index("s") * nc + lax.axis_index("c")
    base = wid * _TPW
    for k in range(4):
        pltpu.sync_copy(idx_hbm.at[k, pl.ds(base, _TPW)], idx_v.at[k])
    for p in range(2):
        pltpu.sync_copy(sc_hbm.at[p, pl.ds(base, _TPW)], sc_v.at[p])
    pltpu.async_copy(values_hbm.at[idx_v.at[0, 0]], buf0, sem0)
    bufs = (buf0, buf1)
    sems = (sem0, sem1)

    @pl.loop(0, _TPW)
    def _token(t):
        acc = tuple(jnp.zeros((16,), jnp.float32) for _ in range(16))
        for kc in range(4):
            nb = bufs[(kc + 1) % 2]
            ns = sems[(kc + 1) % 2]
            if kc < 3:
                pltpu.async_copy(values_hbm.at[idx_v.at[kc + 1, t]], nb, ns)
            else:
                @pl.when(t < _TPW - 1)
                def _():
                    pltpu.async_copy(
                        values_hbm.at[idx_v.at[0, t + 1]], nb, ns)
            cb = bufs[kc % 2]
            pltpu.make_async_copy(
                values_hbm.at[idx_v.at[kc, t]], cb, sems[kc % 2]).wait()
            acc = _acc_chunk(cb, acc)
        for c in range(16):
            res_v[t, pl.ds(c * 16, 16)] = \
                acc[c] * sc_v[c // 8, t, pl.ds((c % 8) * 16, 16)]

    pltpu.sync_copy(res_v, out_hbm.at[pl.ds(base, _TPW)])


def _sc_gather(values_t, idx, scores):
    mesh = plsc.VectorSubcoreMesh(core_axis_name="c", subcore_axis_name="s")
    f32 = jnp.float32
    call = pl.kernel(
        _sc_body,
        out_type=jax.ShapeDtypeStruct((_BS, _VD), f32),
        mesh=mesh,
        scratch_types=[
            pltpu.VMEM((4, _TPW, 128), jnp.int32),
            pltpu.VMEM((2, _TPW, 128), f32),
            pltpu.VMEM((128, 128), f32),
            pltpu.VMEM((128, 128), f32),
            pltpu.VMEM((_TPW, _VD), f32),
            pltpu.SemaphoreType.DMA,
            pltpu.SemaphoreType.DMA,
        ],
        compiler_params=pltpu.CompilerParams(use_tc_tiling_on_sc=False),
    )
    return call(values_t, idx, scores)


def kernel(x, W_q, b_q, keys, values):
    bq2 = b_q.reshape(_HEADS, 1, _KD)
    p1 = jnp.asarray(_P1)
    p2 = jnp.asarray(_P2)
    bias = jnp.asarray(_BIAS)
    sc, idx = _tc_topk(x, W_q, bq2, keys, p1, p2, bias)
    values_t = values.reshape(_NK * _NK // 8, 8, 2, 128) \
        .transpose(0, 2, 1, 3).reshape(_NK * _NK * 2, 128)
    return _sc_gather(values_t, idx, sc)


# speedup vs baseline: 1.0027x; 1.0027x over previous
import functools
import math

import numpy as np
import jax
import jax.numpy as jnp
from jax import lax
from jax.experimental import pallas as pl
from jax.experimental.pallas import tpu as pltpu
from jax.experimental.pallas import tpu_sc as plsc

_HEADS = 8
_KD = 512
_HALF = 256
_NK = 512
_KNN = 32
_IN = 1024
_VD = 256
_BS = 2048
_TB = 256
_NB = _BS // _TB
_NCAND = 128
_NW = 32
_TPW = _BS // _NW


def _build_cand_tables():
    ci, cj = [], []
    for i in range(_KNN):
        for j in range(min(_KNN, _KNN // (i + 1))):
            ci.append(i)
            cj.append(j)
    n = len(ci)
    assert n <= _NCAND
    p1 = np.zeros((_KNN, _NCAND), np.float32)
    p2 = np.zeros((_KNN, _NCAND), np.float32)
    for c in range(n):
        p1[ci[c], c] = 1.0
        p2[cj[c], c] = 1.0
    bias = np.zeros((_NCAND, 1), np.float32)
    bias[n:, 0] = -1e30
    return p1, p2, bias


_P1, _P2, _BIAS = _build_cand_tables()


def _topk32_t(s, src=None):
    c, t = s.shape
    io = lax.broadcasted_iota(jnp.int32, (c, t), 0)
    lane = lax.broadcasted_iota(jnp.int32, (_KNN, t), 0)
    outv = jnp.zeros((_KNN, t), jnp.float32)
    outi = jnp.zeros((_KNN, t), jnp.float32)
    neg = jnp.float32(float("-inf"))
    for k in range(_KNN):
        m = jnp.max(s, axis=0, keepdims=True)
        am = jnp.min(jnp.where(s == m, io, c), axis=0, keepdims=True)
        hit = io == am
        if src is None:
            pick = am.astype(jnp.float32)
        else:
            pick = jnp.sum(jnp.where(hit, src, 0.0), axis=0, keepdims=True)
        outv = jnp.where(lane == k, m, outv)
        outi = jnp.where(lane == k, pick, outi)
        s = jnp.where(hit, neg, s)
    return outv, outi


def _tc_body(x_ref, wq_ref, bq_ref, keys_ref, p1_ref, p2_ref, bias_ref,
             eye_ref, sc_out_ref, idx_out_ref, sc_scr, idx_scr):
    h = pl.program_id(1)
    hi = lax.Precision.HIGHEST
    q = jnp.dot(x_ref[...], wq_ref[...],
                preferred_element_type=jnp.float32)
    q = q + bq_ref[0]
    k1 = keys_ref[0, 0]
    k2 = keys_ref[0, 1]
    dn = (((1,), (1,)), ((), ()))
    dt = (((1,), (1,)), ((), ()))
    s1t = lax.dot_general(k1, q[:, :_HALF], dt,
                          preferred_element_type=jnp.float32)
    s2t = lax.dot_general(k2, q[:, _HALF:], dt,
                          preferred_element_type=jnp.float32)
    v1t, i1t = _topk32_t(s1t)
    v2t, i2t = _topk32_t(s2t)
    p1 = p1_ref[...]
    p2 = p2_ref[...]
    d0 = (((0,), (0,)), ((), ()))
    cst = lax.dot_general(p1, v1t, d0, precision=hi,
                          preferred_element_type=jnp.float32) \
        + lax.dot_general(p2, v2t, d0, precision=hi,
                          preferred_element_type=jnp.float32) \
        + bias_ref[...]
    cidxt = lax.dot_general(p1, i1t, d0, precision=hi,
                            preferred_element_type=jnp.float32) \
        * jnp.float32(_NK) \
        + lax.dot_general(p2, i2t, d0, precision=hi,
                          preferred_element_type=jnp.float32)
    svt, sit = _topk32_t(cst, src=cidxt)
    e = jnp.exp(svt - svt[0:1])
    smt = e / jnp.sum(e, axis=0, keepdims=True)
    eye = eye_ref[...]
    sm = lax.dot_general(smt, eye, d0, precision=hi,
                         preferred_element_type=jnp.float32)
    si = lax.dot_general(sit, eye, d0, precision=hi,
                         preferred_element_type=jnp.float32)
    sc_scr[h] = sm
    idx_scr[h] = (si + jnp.float32(0.5)).astype(jnp.int32)

    @pl.when(h == _HEADS - 1)
    def _flush():
        sc_all = jnp.concatenate([sc_scr[i] for i in range(_HEADS)], axis=1)
        idx_all = jnp.concatenate([idx_scr[i] for i in range(_HEADS)], axis=1)
        sc_out_ref[...] = sc_all.reshape(2 * _TB, 128)
        idx_out_ref[...] = idx_all.reshape(2 * _TB, 128)


def _tc_topk(x, w_q, bq2, keys, p1, p2, bias):
    f32 = jnp.float32
    return pl.pallas_call(
        _tc_body,
        grid=(_NB, _HEADS),
        in_specs=[
            pl.BlockSpec((_TB, _IN), lambda b, h: (b, 0)),
            pl.BlockSpec((_IN, _KD), lambda b, h: (0, h)),
            pl.BlockSpec((1, 1, _KD), lambda b, h: (h, 0, 0)),
            pl.BlockSpec((1, 2, _NK, _HALF), lambda b, h: (h, 0, 0, 0)),
            pl.BlockSpec((_KNN, _NCAND), lambda b, h: (0, 0)),
            pl.BlockSpec((_KNN, _NCAND), lambda b, h: (0, 0)),
            pl.BlockSpec((_NCAND, 1), lambda b, h: (0, 0)),
            pl.BlockSpec((_KNN, _KNN), lambda b, h: (0, 0)),
        ],
        out_specs=[
            pl.BlockSpec((2 * _TB, 128), lambda b, h: (b, 0)),
            pl.BlockSpec((2 * _TB, 128), lambda b, h: (b, 0)),
        ],
        out_shape=[
            jax.ShapeDtypeStruct((_BS * 2, 128), f32),
            jax.ShapeDtypeStruct((_BS * 2, 128), jnp.int32),
        ],
        scratch_shapes=[
            pltpu.VMEM((_HEADS, _TB, _KNN), f32),
            pltpu.VMEM((_HEADS, _TB, _KNN), jnp.int32),
        ],
    )(x, w_q, bq2, keys, p1, p2, bias, jnp.eye(_KNN, dtype=f32))


def _acc_chunk(buf, acc, half):
    c0 = half * 8

    def body(r, a):
        r4 = r * 4
        for rr in range(4):
            a = tuple(
                (a[c] + buf[r4 + rr, pl.ds((c - c0) * 16, 16)])
                if c0 <= c < c0 + 8 else a[c]
                for c in range(16)
            )
        return a
    return lax.fori_loop(0, 128 // 4, body, acc)


def _sc_body(values_hbm, idx_hbm, sc_hbm, out_hbm, idx_v, sc_v, buf0, buf1,
             res_v, pidx_v, sem0, sem1):
    nc = 2
    wid = lax.axis_index("s") * nc + lax.axis_index("c")
    base = wid * _TPW
    pltpu.sync_copy(idx_hbm.at[pl.ds(base * 2, _TPW * 2)], idx_v)
    pltpu.sync_copy(sc_hbm.at[pl.ds(base * 2, _TPW * 2)], sc_v)

    def _conv(r, _):
        for c in range(8):
            i = idx_v[r, pl.ds(c * 16, 16)]
            p0 = i + (i & jnp.int32(-8))
            pidx_v[2 * r, pl.ds(c * 16, 16)] = p0
            pidx_v[2 * r + 1, pl.ds(c * 16, 16)] = p0 + jnp.int32(8)
        return 0
    lax.fori_loop(0, _TPW * 2, _conv, 0)

    pltpu.async_copy(values_hbm.at[pidx_v.at[0]], buf0, sem0)
    bufs = (buf0, buf1)
    sems = (sem0, sem1)

    @pl.loop(0, _TPW)
    def _token(t):
        g = t * 4
        acc = tuple(jnp.zeros((16,), jnp.float32) for _ in range(16))
        for kc in range(4):
            nb = bufs[(kc + 1) % 2]
            ns = sems[(kc + 1) % 2]
            if kc < 3:
                pltpu.async_copy(values_hbm.at[pidx_v.at[g + kc + 1]], nb, ns)
            else:
                @pl.when(t < _TPW - 1)
                def _():
                    pltpu.async_copy(
                        values_hbm.at[pidx_v.at[g + 4]], nb, ns)
            cb = bufs[kc % 2]
            pltpu.make_async_copy(
                values_hbm.at[pidx_v.at[g + kc]], cb, sems[kc % 2]).wait()
            acc = _acc_chunk(cb, acc, kc % 2)
        for c in range(16):
            res_v[t, pl.ds(c * 16, 16)] = \
                acc[c] * sc_v[t * 2 + c // 8, pl.ds((c % 8) * 16, 16)]

    pltpu.sync_copy(res_v, out_hbm.at[pl.ds(base, _TPW)])


def _sc_gather(values_t, idx, scores):
    mesh = plsc.VectorSubcoreMesh(core_axis_name="c", subcore_axis_name="s")
    f32 = jnp.float32
    call = pl.kernel(
        _sc_body,
        out_type=jax.ShapeDtypeStruct((_BS, _VD), f32),
        mesh=mesh,
        scratch_types=[
            pltpu.VMEM((_TPW * 2, 128), jnp.int32),
            pltpu.VMEM((_TPW * 2, 128), f32),
            pltpu.VMEM((128, 128), f32),
            pltpu.VMEM((128, 128), f32),
            pltpu.VMEM((_TPW, _VD), f32),
            pltpu.VMEM((_TPW * 4, 128), jnp.int32),
            pltpu.SemaphoreType.DMA,
            pltpu.SemaphoreType.DMA,
        ],
        compiler_params=pltpu.CompilerParams(use_tc_tiling_on_sc=False),
    )
    return call(values_t, idx, scores)


def kernel(x, W_q, b_q, keys, values):
    bq2 = b_q.reshape(_HEADS, 1, _KD)
    p1 = jnp.asarray(_P1)
    p2 = jnp.asarray(_P2)
    bias = jnp.asarray(_BIAS)
    sc, idx = _tc_topk(x, W_q, bq2, keys, p1, p2, bias)
    values_t = values.reshape(_NK * _NK // 8, 8, 2, 128) \
        .transpose(0, 2, 1, 3).reshape(_NK * _NK * 2, 128)
    return _sc_gather(values_t, idx, sc)


# speedup vs baseline: 1.0531x; 1.0503x over previous
import functools
import math

import numpy as np
import jax
import jax.numpy as jnp
from jax import lax
from jax.experimental import pallas as pl
from jax.experimental.pallas import tpu as pltpu
from jax.experimental.pallas import tpu_sc as plsc

_HEADS = 8
_KD = 512
_HALF = 256
_NK = 512
_KNN = 32
_IN = 1024
_VD = 256
_BS = 2048
_TB = 256
_NB = _BS // _TB
_NCAND = 128
_NW = 32
_TPW = _BS // _NW


def _build_cand_tables():
    ci, cj = [], []
    for i in range(_KNN):
        for j in range(min(_KNN, _KNN // (i + 1))):
            ci.append(i)
            cj.append(j)
    n = len(ci)
    assert n <= _NCAND
    p1 = np.zeros((_KNN, _NCAND), np.float32)
    p2 = np.zeros((_KNN, _NCAND), np.float32)
    for c in range(n):
        p1[ci[c], c] = 1.0
        p2[cj[c], c] = 1.0
    bias = np.zeros((_NCAND, 1), np.float32)
    bias[n:, 0] = -1e30
    return p1, p2, bias


_P1, _P2, _BIAS = _build_cand_tables()


def _topk32_t(s, src=None):
    c, t = s.shape
    io = lax.broadcasted_iota(jnp.int32, (c, t), 0)
    lane = lax.broadcasted_iota(jnp.int32, (_KNN, t), 0)
    outv = jnp.zeros((_KNN, t), jnp.float32)
    outi = jnp.zeros((_KNN, t), jnp.float32)
    neg = jnp.float32(float("-inf"))
    for k in range(_KNN):
        m = jnp.max(s, axis=0, keepdims=True)
        am = jnp.min(jnp.where(s == m, io, c), axis=0, keepdims=True)
        hit = io == am
        if src is None:
            pick = am.astype(jnp.float32)
        else:
            pick = jnp.sum(jnp.where(hit, src, 0.0), axis=0, keepdims=True)
        outv = jnp.where(lane == k, m, outv)
        outi = jnp.where(lane == k, pick, outi)
        s = jnp.where(hit, neg, s)
    return outv, outi


def _tc_body(x_ref, wq_ref, bq_ref, keys_ref, p1_ref, p2_ref, bias_ref,
             eye_ref, sc_out_ref, idx_out_ref, sc_scr, idx_scr):
    h = pl.program_id(1)
    hi = lax.Precision.HIGHEST
    q = jnp.dot(x_ref[...], wq_ref[...],
                preferred_element_type=jnp.float32)
    q = q + bq_ref[0]
    k1 = keys_ref[0, 0]
    k2 = keys_ref[0, 1]
    dn = (((1,), (1,)), ((), ()))
    dt = (((1,), (1,)), ((), ()))
    s1t = lax.dot_general(k1, q[:, :_HALF], dt,
                          preferred_element_type=jnp.float32)
    s2t = lax.dot_general(k2, q[:, _HALF:], dt,
                          preferred_element_type=jnp.float32)
    v1t, i1t = _topk32_t(s1t)
    v2t, i2t = _topk32_t(s2t)
    p1 = p1_ref[...]
    p2 = p2_ref[...]
    d0 = (((0,), (0,)), ((), ()))
    cst = lax.dot_general(p1, v1t, d0, precision=hi,
                          preferred_element_type=jnp.float32) \
        + lax.dot_general(p2, v2t, d0, precision=hi,
                          preferred_element_type=jnp.float32) \
        + bias_ref[...]
    cidxt = lax.dot_general(p1, i1t, d0, precision=hi,
                            preferred_element_type=jnp.float32) \
        * jnp.float32(_NK) \
        + lax.dot_general(p2, i2t, d0, precision=hi,
                          preferred_element_type=jnp.float32)
    svt, sit = _topk32_t(cst, src=cidxt)
    e = jnp.exp(svt - svt[0:1])
    smt = e / jnp.sum(e, axis=0, keepdims=True)
    eye = eye_ref[...]
    sm = lax.dot_general(smt, eye, d0, precision=hi,
                         preferred_element_type=jnp.float32)
    si = lax.dot_general(sit, eye, d0, precision=hi,
                         preferred_element_type=jnp.float32)
    sc_scr[h] = sm
    idx_scr[h] = (si + jnp.float32(0.5)).astype(jnp.int32)

    @pl.when(h == _HEADS - 1)
    def _flush():
        sc_all = jnp.concatenate([sc_scr[i] for i in range(_HEADS)], axis=1)
        idx_all = jnp.concatenate([idx_scr[i] for i in range(_HEADS)], axis=1)
        sc_out_ref[...] = sc_all.reshape(2 * _TB, 128)
        idx_out_ref[...] = idx_all.reshape(2 * _TB, 128)


def _tc_topk(x, w_q, bq2, keys, p1, p2, bias):
    f32 = jnp.float32
    return pl.pallas_call(
        _tc_body,
        grid=(_NB, _HEADS),
        in_specs=[
            pl.BlockSpec((_TB, _IN), lambda b, h: (b, 0)),
            pl.BlockSpec((_IN, _KD), lambda b, h: (0, h)),
            pl.BlockSpec((1, 1, _KD), lambda b, h: (h, 0, 0)),
            pl.BlockSpec((1, 2, _NK, _HALF), lambda b, h: (h, 0, 0, 0)),
            pl.BlockSpec((_KNN, _NCAND), lambda b, h: (0, 0)),
            pl.BlockSpec((_KNN, _NCAND), lambda b, h: (0, 0)),
            pl.BlockSpec((_NCAND, 1), lambda b, h: (0, 0)),
            pl.BlockSpec((_KNN, _KNN), lambda b, h: (0, 0)),
        ],
        out_specs=[
            pl.BlockSpec((2 * _TB, 128), lambda b, h: (b, 0)),
            pl.BlockSpec((2 * _TB, 128), lambda b, h: (b, 0)),
        ],
        out_shape=[
            jax.ShapeDtypeStruct((_BS * 2, 128), f32),
            jax.ShapeDtypeStruct((_BS * 2, 128), jnp.int32),
        ],
        scratch_shapes=[
            pltpu.VMEM((_HEADS, _TB, _KNN), f32),
            pltpu.VMEM((_HEADS, _TB, _KNN), jnp.int32),
        ],
    )(x, w_q, bq2, keys, p1, p2, bias, jnp.eye(_KNN, dtype=f32))


def _acc_chunk(buf, acc):
    def body(r, a):
        r4 = r * 4
        for rr in range(4):
            a = tuple(
                a[c] + buf[r4 + rr, pl.ds(c * 16, 16)] for c in range(16)
            )
        return a
    return lax.fori_loop(0, 128 // 4, body, acc)


def _sc_body(values_hbm, idx_hbm, sc_hbm, out_hbm, idx_v, sc_v, buf0, buf1,
             res_v, sem0, sem1):
    nc = 2
    wid = lax.axis_index("s") * nc + lax.axis_index("c")
    base = wid * _TPW
    pltpu.sync_copy(idx_hbm.at[pl.ds(base * 2, _TPW * 2)], idx_v)
    pltpu.sync_copy(sc_hbm.at[pl.ds(base * 2, _TPW * 2)], sc_v)
    pltpu.async_copy(values_hbm.at[idx_v.at[0]], buf0, sem0)

    @pl.loop(0, _TPW)
    def _token(t):
        g = t * 2
        pltpu.async_copy(values_hbm.at[idx_v.at[g + 1]], buf1, sem1)
        pltpu.make_async_copy(values_hbm.at[idx_v.at[g]], buf0, sem0).wait()
        acc0 = tuple(jnp.zeros((16,), jnp.float32) for _ in range(16))
        acc = _acc_chunk(buf0, acc0)

        @pl.when(t < _TPW - 1)
        def _():
            pltpu.async_copy(values_hbm.at[idx_v.at[g + 2]], buf0, sem0)

        pltpu.make_async_copy(values_hbm.at[idx_v.at[g + 1]], buf1, sem1).wait()
        acc = _acc_chunk(buf1, acc)
        for c in range(16):
            res_v[t, pl.ds(c * 16, 16)] = \
                acc[c] * sc_v[g + c // 8, pl.ds((c % 8) * 16, 16)]

    pltpu.sync_copy(res_v, out_hbm.at[pl.ds(base, _TPW)])


def _sc_gather(values, idx, scores):
    mesh = plsc.VectorSubcoreMesh(core_axis_name="c", subcore_axis_name="s")
    f32 = jnp.float32
    call = pl.kernel(
        _sc_body,
        out_type=jax.ShapeDtypeStruct((_BS, _VD), f32),
        mesh=mesh,
        scratch_types=[
            pltpu.VMEM((_TPW * 2, 128), jnp.int32),
            pltpu.VMEM((_TPW * 2, 128), f32),
            pltpu.VMEM((128, _VD), f32),
            pltpu.VMEM((128, _VD), f32),
            pltpu.VMEM((_TPW, _VD), f32),
            pltpu.SemaphoreType.DMA,
            pltpu.SemaphoreType.DMA,
        ],
        compiler_params=pltpu.CompilerParams(use_tc_tiling_on_sc=False),
    )
    return call(values, idx, scores)


def kernel(x, W_q, b_q, keys, values):
    bq2 = b_q.reshape(_HEADS, 1, _KD)
    p1 = jnp.asarray(_P1)
    p2 = jnp.asarray(_P2)
    bias = jnp.asarray(_BIAS)
    sc, idx = _tc_topk(x, W_q, bq2, keys, p1, p2, bias)
    return _sc_gather(values, idx, sc)
